# Initial kernel scaffold; baseline (speedup 1.0000x reference)
import jax
import jax.numpy as jnp
from jax.experimental import pallas as pl


def kernel(X, vertex, edges, W, degE, degV):
    raise NotImplementedError("write your pallas kernel here")


# speedup vs baseline: 3.2747x; 3.2747x over previous
import functools

import jax
import jax.numpy as jnp
from jax import lax
from jax.experimental import pallas as pl
from jax.experimental.pallas import tpu as pltpu
from jax.experimental.pallas import tpu_sc as plsc

N = 10000
NNZ = 160000
E = 5000
D_IN = 256
D_HID = 256
HALF = 128

NT = 16
NC = 2
CHUNK = 128
CHUNKS_PER_TILE = 80
PAIRS_PER_TILE = CHUNK * CHUNKS_PER_TILE
NNZ_PAD = PAIRS_PER_TILE * NT

E_PAD = 5120
N_PAD = 10112
E_PER_TILE = E_PAD // NT
NV_PER_TILE = N_PAD // NT


def _mm_body(x_ref, w_ref, o_ref):
    o_ref[0] = jnp.dot(x_ref[...], w_ref[...],
                       preferred_element_type=jnp.float32)


def _matmul_split(X, W):
    return pl.pallas_call(
        _mm_body,
        grid=(5, NC),
        in_specs=[
            pl.BlockSpec((2000, D_IN), lambda i, c: (i, 0)),
            pl.BlockSpec((D_IN, HALF), lambda i, c: (0, c)),
        ],
        out_specs=pl.BlockSpec((1, 2000, HALF), lambda i, c: (c, i, 0)),
        out_shape=jax.ShapeDtypeStruct((NC, N, HALF), jnp.float32),
    )(X, W)


def _norm_body(xv_ref, dv_ref, o_ref):
    a = xv_ref[0] * dv_ref[...]
    b = xv_ref[1] * dv_ref[...]
    ss = (jnp.sum(a * a, axis=1, keepdims=True)
          + jnp.sum(b * b, axis=1, keepdims=True))
    rn = jnp.sqrt(ss)
    sc = jnp.where(rn > 0, 1.0 / rn, 0.0)
    o_ref[:, :HALF] = a * sc
    o_ref[:, HALF:] = b * sc


def _normalize(xv_split, degV):
    return pl.pallas_call(
        _norm_body,
        grid=(5,),
        in_specs=[
            pl.BlockSpec((NC, 2000, HALF), lambda i: (0, i, 0)),
            pl.BlockSpec((2000, 1), lambda i: (i, 0)),
        ],
        out_specs=pl.BlockSpec((2000, D_HID), lambda i: (i, 0)),
        out_shape=jax.ShapeDtypeStruct((N, D_HID), jnp.float32),
    )(xv_split, degV)


def _sc_body(xp_ref, vg_ref, e_ref, vs_ref, dege_ref, zw_ref, z1_ref,
             ones_ref, out_ref, xe_hbm,
             acc_sh, cnt_sh,
             vg_v, e_v, vs_v, gbuf, dc_v, scal_v, ones_v, sem):
    c = lax.axis_index("c")
    sid = lax.axis_index("s")
    wid = c * NT + sid

    pltpu.sync_copy(vg_ref.at[wid], vg_v)
    pltpu.sync_copy(e_ref.at[sid], e_v)
    pltpu.sync_copy(vs_ref.at[sid], vs_v)
    pltpu.sync_copy(ones_ref, ones_v)

    pltpu.sync_copy(zw_ref.at[pl.ds(0, E_PER_TILE)],
                    acc_sh.at[pl.ds(sid * E_PER_TILE, E_PER_TILE)])
    pltpu.sync_copy(z1_ref, dc_v)
    pltpu.sync_copy(dc_v,
                    cnt_sh.at[pl.ds(sid * E_PER_TILE, E_PER_TILE)])
    plsc.subcore_barrier()

    @pl.loop(0, CHUNKS_PER_TILE)
    def _phase_a(j):
        pltpu.async_copy(xp_ref.at[vg_v.at[j]], gbuf, sem).wait()
        pltpu.sync_copy(gbuf, acc_sh.at[e_v.at[j]], add=True)
        pltpu.sync_copy(ones_v, cnt_sh.at[e_v.at[j]], add=True)

    plsc.subcore_barrier()

    base = sid * E_PER_TILE
    pltpu.sync_copy(dege_ref.at[pl.ds(base, E_PER_TILE)], dc_v)
    sbuf = gbuf.at[pl.ds(0, 16)]

    @pl.loop(0, E_PER_TILE // 16)
    def _scale(jj):
        row0 = base + jj * 16
        pltpu.sync_copy(acc_sh.at[pl.ds(row0, 16)], sbuf)
        pltpu.sync_copy(cnt_sh.at[pl.ds(row0, 16)], scal_v)
        cvec = scal_v[...]
        dvec = dc_v[pl.ds(jj * 16, 16)]
        svec = dvec / jnp.maximum(cvec, 1.0)
        for r in range(16):
            s = svec[r]
            for k in range(HALF // 16):
                sbuf[r, pl.ds(k * 16, 16)] = sbuf[r, pl.ds(k * 16, 16)] * s
        pltpu.sync_copy(sbuf, acc_sh.at[pl.ds(row0, 16)])

    plsc.subcore_barrier()
    pltpu.sync_copy(acc_sh.at[pl.ds(base, E_PER_TILE)],
                    xe_hbm.at[pl.ds(c * E_PAD + base, E_PER_TILE)])
    plsc.subcore_barrier()

    pltpu.sync_copy(zw_ref.at[pl.ds(0, NV_PER_TILE)],
                    acc_sh.at[pl.ds(sid * NV_PER_TILE, NV_PER_TILE)])
    off = c * E_PAD

    @pl.loop(0, CHUNKS_PER_TILE)
    def _offset(j):
        for k in range(CHUNK // 16):
            sl = pl.ds(k * 16, 16)
            e_v[j, sl] = e_v[j, sl] + off

    plsc.subcore_barrier()

    @pl.loop(0, CHUNKS_PER_TILE)
    def _phase_b(j):
        pltpu.async_copy(xe_hbm.at[e_v.at[j]], gbuf, sem).wait()
        pltpu.sync_copy(gbuf, acc_sh.at[vs_v.at[j]], add=True)

    plsc.subcore_barrier()

    out0 = sid * NV_PER_TILE
    pltpu.sync_copy(acc_sh.at[pl.ds(out0, NV_PER_TILE)],
                    out_ref.at[pl.ds(c * N_PAD + out0, NV_PER_TILE)])


def _sc_aggregate(xp_flat, vg2, e_t, vs_t, degE_pad, zeros_w, zeros_1,
                  ones_c):
    mesh = plsc.VectorSubcoreMesh(core_axis_name="c", subcore_axis_name="s")
    f = pl.kernel(
        _sc_body,
        out_type=(jax.ShapeDtypeStruct((NC * N_PAD, HALF), jnp.float32),
                  jax.ShapeDtypeStruct((NC * E_PAD, HALF), jnp.float32)),
        mesh=mesh,
        scratch_types=[
            pltpu.VMEM_SHARED((N_PAD, HALF), jnp.float32),
            pltpu.VMEM_SHARED((E_PAD,), jnp.float32),
            pltpu.VMEM((CHUNKS_PER_TILE, CHUNK), jnp.int32),
            pltpu.VMEM((CHUNKS_PER_TILE, CHUNK), jnp.int32),
            pltpu.VMEM((CHUNKS_PER_TILE, CHUNK), jnp.int32),
            pltpu.VMEM((CHUNK, HALF), jnp.float32),
            pltpu.VMEM((E_PER_TILE,), jnp.float32),
            pltpu.VMEM((16,), jnp.float32),
            pltpu.VMEM((CHUNK,), jnp.float32),
            pltpu.SemaphoreType.DMA,
        ],
    )
    out, _ = f(xp_flat, vg2, e_t, vs_t, degE_pad, zeros_w, zeros_1, ones_c)
    return out


@jax.jit
def kernel(X, vertex, edges, W, degE, degV):
    xp = _matmul_split(X, W)
    xp_flat = xp.reshape(NC * N, HALF)

    pad = NNZ_PAD - NNZ
    vg = jnp.concatenate([vertex, jnp.zeros((pad,), jnp.int32)])
    e_p = jnp.concatenate([edges, jnp.full((pad,), E, jnp.int32)])
    vs = jnp.concatenate([vertex, jnp.full((pad,), N, jnp.int32)])
    vg_t = vg.reshape(NT, CHUNKS_PER_TILE, CHUNK)
    vg2 = jnp.concatenate([vg_t, vg_t + N], axis=0)
    e_t = e_p.reshape(NT, CHUNKS_PER_TILE, CHUNK)
    vs_t = vs.reshape(NT, CHUNKS_PER_TILE, CHUNK)

    degE_pad = jnp.concatenate(
        [degE[:, 0], jnp.ones((E_PAD - E,), jnp.float32)])
    zeros_w = jnp.zeros((NV_PER_TILE, HALF), jnp.float32)
    zeros_1 = jnp.zeros((E_PER_TILE,), jnp.float32)
    ones_c = jnp.ones((CHUNK,), jnp.float32)

    xv_flat = _sc_aggregate(xp_flat, vg2, e_t, vs_t,
                            degE_pad, zeros_w, zeros_1, ones_c)
    xv_split = xv_flat.reshape(NC, N_PAD, HALF)[:, :N]
    return _normalize(xv_split, degV)


# speedup vs baseline: 3.9364x; 1.2021x over previous
import jax
import jax.numpy as jnp
from jax import lax
from jax.experimental import pallas as pl
from jax.experimental.pallas import tpu as pltpu
from jax.experimental.pallas import tpu_sc as plsc

N = 10000
NNZ = 160000
E = 5000
D_IN = 256
D_HID = 256
HALF = 128

NT = 16
NC = 2
CHUNK = 128
NCH = 80
PAIRS_PER_TILE = CHUNK * NCH
NNZ_PAD = PAIRS_PER_TILE * NT

E_PAD = 5120
N_PAD = 10112
E_PER_TILE = E_PAD // NT
NV_PER_TILE = N_PAD // NT


def _mm_body(x_ref, w_ref, o_ref):
    o_ref[0] = jnp.dot(x_ref[...], w_ref[...],
                       preferred_element_type=jnp.float32)


def _matmul_split(X, W):
    return pl.pallas_call(
        _mm_body,
        grid=(5, NC),
        in_specs=[
            pl.BlockSpec((2000, D_IN), lambda i, c: (i, 0)),
            pl.BlockSpec((D_IN, HALF), lambda i, c: (0, c)),
        ],
        out_specs=pl.BlockSpec((1, 2000, HALF), lambda i, c: (c, i, 0)),
        out_shape=jax.ShapeDtypeStruct((NC, N, HALF), jnp.float32),
    )(X, W)


def _norm_body(xv_ref, dv_ref, o_ref):
    a = xv_ref[0] * dv_ref[...]
    b = xv_ref[1] * dv_ref[...]
    ss = (jnp.sum(a * a, axis=1, keepdims=True)
          + jnp.sum(b * b, axis=1, keepdims=True))
    rn = jnp.sqrt(ss)
    sc = jnp.where(rn > 0, 1.0 / rn, 0.0)
    o_ref[:, :HALF] = a * sc
    o_ref[:, HALF:] = b * sc


def _normalize(xv_split, degV):
    return pl.pallas_call(
        _norm_body,
        grid=(5,),
        in_specs=[
            pl.BlockSpec((NC, 2000, HALF), lambda i: (0, i, 0)),
            pl.BlockSpec((2000, 1), lambda i: (i, 0)),
        ],
        out_specs=pl.BlockSpec((2000, D_HID), lambda i: (i, 0)),
        out_shape=jax.ShapeDtypeStruct((N, D_HID), jnp.float32),
    )(xv_split, degV)


def _pipeline(src_ref, gi_ref, si_ref, row_sel, scatter_fn,
              g_ring, s_ring, gbuf, gsem0, gsem1, isem0, isem1):
    gsel, ssel = row_sel
    b0, b1 = gbuf.at[0], gbuf.at[1]

    def idx_copy(j, slot, sem, async_=True):
        a = pltpu.async_copy if async_ else None
        if async_:
            pltpu.async_copy(gi_ref.at[gsel, j], g_ring.at[slot], sem)
            pltpu.async_copy(si_ref.at[ssel, j], s_ring.at[slot], sem)
        else:
            pltpu.sync_copy(gi_ref.at[gsel, j], g_ring.at[slot])
            pltpu.sync_copy(si_ref.at[ssel, j], s_ring.at[slot])

    def idx_wait(j, slot, sem):
        pltpu.make_async_copy(gi_ref.at[gsel, j], g_ring.at[slot], sem).wait()
        pltpu.make_async_copy(si_ref.at[ssel, j], s_ring.at[slot], sem).wait()

    def gather(slot, buf, sem):
        pltpu.async_copy(src_ref.at[g_ring.at[slot]], buf, sem)

    def gather_wait(slot, buf, sem):
        pltpu.make_async_copy(src_ref.at[g_ring.at[slot]], buf, sem).wait()

    idx_copy(0, 0, isem0, async_=False)
    gather(0, b0, gsem0)
    idx_copy(1, 1, isem1)

    @pl.loop(0, NCH // 2)
    def _body(i):
        j = i * 2
        idx_wait(j + 1, 1, isem1)
        gather(1, b1, gsem1)
        gather_wait(0, b0, gsem0)
        scatter_fn(b0, s_ring.at[0])

        @pl.when(i < NCH // 2 - 1)
        def _refill0():
            idx_copy(j + 2, 0, isem0)
            idx_wait(j + 2, 0, isem0)
            gather(0, b0, gsem0)

        gather_wait(1, b1, gsem1)
        scatter_fn(b1, s_ring.at[1])

        @pl.when(i < NCH // 2 - 1)
        def _refill1():
            idx_copy(j + 3, 1, isem1)


def _sc_body(xp_ref, vga_ref, ea_ref, eb_ref, vs_ref, dege_ref, zw_ref,
             z1_ref, ones_ref, out_ref, xe_hbm,
             acc_sh, cnt_sh,
             g_ring, s_ring, gbuf, dc_v, scal_v, ones_v,
             gsem0, gsem1, isem0, isem1):
    c = lax.axis_index("c")
    sid = lax.axis_index("s")
    wid = c * NT + sid

    pltpu.sync_copy(ones_ref, ones_v)

    pltpu.sync_copy(zw_ref.at[pl.ds(0, E_PER_TILE)],
                    acc_sh.at[pl.ds(sid * E_PER_TILE, E_PER_TILE)])
    pltpu.sync_copy(z1_ref, dc_v)
    pltpu.sync_copy(dc_v,
                    cnt_sh.at[pl.ds(sid * E_PER_TILE, E_PER_TILE)])
    plsc.subcore_barrier()

    def scatter_a(buf, sidx):
        pltpu.sync_copy(buf, acc_sh.at[sidx], add=True)
        pltpu.sync_copy(ones_v, cnt_sh.at[sidx], add=True)

    _pipeline(xp_ref, vga_ref, ea_ref, (wid, sid), scatter_a,
              g_ring, s_ring, gbuf, gsem0, gsem1, isem0, isem1)
    plsc.subcore_barrier()

    base = sid * E_PER_TILE
    pltpu.sync_copy(dege_ref.at[pl.ds(base, E_PER_TILE)], dc_v)
    sbuf = gbuf.at[0, pl.ds(0, 16)]

    @pl.loop(0, E_PER_TILE // 16)
    def _scale(jj):
        row0 = base + jj * 16
        pltpu.sync_copy(acc_sh.at[pl.ds(row0, 16)], sbuf)
        pltpu.sync_copy(cnt_sh.at[pl.ds(row0, 16)], scal_v)
        cvec = scal_v[...]
        dvec = dc_v[pl.ds(jj * 16, 16)]
        svec = dvec / jnp.maximum(cvec, 1.0)
        for r in range(16):
            s = svec[r]
            for k in range(HALF // 16):
                sbuf[r, pl.ds(k * 16, 16)] = sbuf[r, pl.ds(k * 16, 16)] * s
        pltpu.sync_copy(sbuf, acc_sh.at[pl.ds(row0, 16)])

    plsc.subcore_barrier()
    pltpu.sync_copy(acc_sh.at[pl.ds(base, E_PER_TILE)],
                    xe_hbm.at[pl.ds(c * E_PAD + base, E_PER_TILE)])
    plsc.subcore_barrier()

    pltpu.sync_copy(zw_ref.at[pl.ds(0, NV_PER_TILE)],
                    acc_sh.at[pl.ds(sid * NV_PER_TILE, NV_PER_TILE)])
    plsc.subcore_barrier()

    def scatter_b(buf, sidx):
        pltpu.sync_copy(buf, acc_sh.at[sidx], add=True)

    _pipeline(xe_hbm, eb_ref, vs_ref, (wid, sid), scatter_b,
              g_ring, s_ring, gbuf, gsem0, gsem1, isem0, isem1)
    plsc.subcore_barrier()

    out0 = sid * NV_PER_TILE
    pltpu.sync_copy(acc_sh.at[pl.ds(out0, NV_PER_TILE)],
                    out_ref.at[pl.ds(c * N_PAD + out0, NV_PER_TILE)])


def _sc_aggregate(xp_flat, vga, ea, eb, vs, degE_pad, zeros_w, zeros_1,
                  ones_c):
    mesh = plsc.VectorSubcoreMesh(core_axis_name="c", subcore_axis_name="s")
    f = pl.kernel(
        _sc_body,
        out_type=(jax.ShapeDtypeStruct((NC * N_PAD, HALF), jnp.float32),
                  jax.ShapeDtypeStruct((NC * E_PAD, HALF), jnp.float32)),
        mesh=mesh,
        scratch_types=[
            pltpu.VMEM_SHARED((N_PAD, HALF), jnp.float32),
            pltpu.VMEM_SHARED((E_PAD,), jnp.float32),
            pltpu.VMEM((2, CHUNK), jnp.int32),
            pltpu.VMEM((2, CHUNK), jnp.int32),
            pltpu.VMEM((2, CHUNK, HALF), jnp.float32),
            pltpu.VMEM((E_PER_TILE,), jnp.float32),
            pltpu.VMEM((16,), jnp.float32),
            pltpu.VMEM((CHUNK,), jnp.float32),
            pltpu.SemaphoreType.DMA,
            pltpu.SemaphoreType.DMA,
            pltpu.SemaphoreType.DMA,
            pltpu.SemaphoreType.DMA,
        ],
    )
    out, _ = f(xp_flat, vga, ea, eb, vs, degE_pad, zeros_w, zeros_1, ones_c)
    return out


@jax.jit
def kernel(X, vertex, edges, W, degE, degV):
    xp = _matmul_split(X, W)
    xp_flat = xp.reshape(NC * N, HALF)

    pad = NNZ_PAD - NNZ
    vg = jnp.concatenate([vertex, jnp.zeros((pad,), jnp.int32)])
    e_p = jnp.concatenate([edges, jnp.full((pad,), E, jnp.int32)])
    vs = jnp.concatenate([vertex, jnp.full((pad,), N, jnp.int32)])
    vg_t = vg.reshape(NT, NCH, CHUNK)
    e_t = e_p.reshape(NT, NCH, CHUNK)
    vs_t = vs.reshape(NT, NCH, CHUNK)
    vga = jnp.concatenate([vg_t, vg_t + N], axis=0)
    eb = jnp.concatenate([e_t, e_t + E_PAD], axis=0)

    degE_pad = jnp.concatenate(
        [degE[:, 0], jnp.ones((E_PAD - E,), jnp.float32)])
    zeros_w = jnp.zeros((NV_PER_TILE, HALF), jnp.float32)
    zeros_1 = jnp.zeros((E_PER_TILE,), jnp.float32)
    ones_c = jnp.ones((CHUNK,), jnp.float32)

    xv_flat = _sc_aggregate(xp_flat, vga, e_t, eb, vs_t,
                            degE_pad, zeros_w, zeros_1, ones_c)
    xv_split = xv_flat.reshape(NC, N_PAD, HALF)[:, :N]
    return _normalize(xv_split, degV)
